# Initial kernel scaffold; baseline (speedup 1.0000x reference)
import jax
import jax.numpy as jnp
from jax.experimental import pallas as pl


def kernel(x, edge_index, batch, atom_W, atom_b, conv_W, conv_b, norm_g, norm_b, fc_W, fc_b, fcn_g, fcn_b, out_W, out_b):
    raise NotImplementedError("write your pallas kernel here")


# speedup vs baseline: 12.1798x; 12.1798x over previous
import functools

import jax
import jax.numpy as jnp
from jax import lax
from jax.experimental import pallas as pl
from jax.experimental.pallas import tpu as pltpu
from jax.experimental.pallas import tpu_sc as plsc

N = 10000
E = 320000
H = 128
NG = 128
NOUT = 2
EPS = 1e-5

NC = 2
NS = 16
NW = NC * NS
HH = H // NC
EPS_T = E // NS
CH = 80
NCH = EPS_T // CH
EPW = E // NW
NCHD = EPW // CH
NP = 10240
RPT = NP // NS
ZR = 128
L = 16
DW = 16


def _deg_body(dst_hbm, degp_hbm, dst_v, ones_v, zb_v, degacc_sh):
    c = lax.axis_index("c")
    s = lax.axis_index("s")
    w = c * NS + s
    pltpu.sync_copy(dst_hbm.at[w], dst_v)

    ones = jnp.ones((L,), jnp.float32)

    def fill_ones(i, carry):
        ones_v[i, pl.ds(0, L)] = ones
        return carry

    lax.fori_loop(0, CH, fill_ones, 0)

    def fill_zero(i, carry):
        zb_v[i, pl.ds(0, L)] = jnp.zeros((L,), jnp.float32)
        return carry

    lax.fori_loop(0, RPT, fill_zero, 0)
    pltpu.sync_copy(zb_v, degacc_sh.at[pl.ds(s * RPT, RPT)])
    plsc.subcore_barrier()

    def chunk(ci, carry):
        pltpu.sync_copy(ones_v, degacc_sh.at[dst_v.at[ci]], add=True)
        return carry

    lax.fori_loop(0, NCHD, chunk, 0)
    plsc.subcore_barrier()
    pltpu.sync_copy(degacc_sh.at[pl.ds(s * RPT, RPT)],
                    degp_hbm.at[c, pl.ds(s * RPT, RPT)])


def _msg_body(hw_hbm, src_hbm, dst_hbm, out_hbm, src_v, dst_v, rows_v, zb_v,
              acc_sh, sem):
    c = lax.axis_index("c")
    s = lax.axis_index("s")

    def zero_row(i, carry):
        for j in range(HH // L):
            zb_v[i, pl.ds(j * L, L)] = jnp.zeros((L,), jnp.float32)
        return carry

    lax.fori_loop(0, ZR, zero_row, 0)
    for j in range(RPT // ZR):
        pltpu.sync_copy(zb_v, acc_sh.at[pl.ds(s * RPT + j * ZR, ZR)])

    pltpu.sync_copy(src_hbm.at[s], src_v)
    pltpu.sync_copy(dst_hbm.at[s], dst_v)
    plsc.subcore_barrier()

    def chunk(ci, carry):
        pltpu.async_copy(hw_hbm.at[c].at[src_v.at[ci]], rows_v, sem).wait()
        pltpu.sync_copy(rows_v, acc_sh.at[dst_v.at[ci]], add=True)
        return carry

    lax.fori_loop(0, NCH, chunk, 0)
    plsc.subcore_barrier()

    pltpu.sync_copy(acc_sh.at[pl.ds(s * RPT, RPT)],
                    out_hbm.at[c, pl.ds(s * RPT, RPT)])


def _elu(v):
    return jnp.where(v > 0, v, jnp.exp(v) - 1.0)


def _bn_relu(t, g, b):
    m = jnp.mean(t, axis=0, keepdims=True)
    v = jnp.mean((t - m) ** 2, axis=0, keepdims=True)
    return jnp.maximum((t - m) * lax.rsqrt(v + EPS) * g + b, 0.0)


def _store_halves(hw_ref, hww):
    hw_ref[0] = hww[:, :HH]
    hw_ref[1] = hww[:, HH:]


def _combine(a, hwp):
    return jnp.concatenate([a[0, :N] + hwp[0], a[1, :N] + hwp[1]], axis=1)


def _k0_body(x_ref, aW_ref, ab_ref, W0_ref, degp_ref, hw_ref, dis_ref):
    dp = degp_ref[...]
    deg = dp[0, :N, 0:1] + dp[1, :N, 0:1] + 1.0
    dis = lax.rsqrt(deg)
    h = _elu(jnp.dot(x_ref[...], aW_ref[...],
                     preferred_element_type=jnp.float32) + ab_ref[...])
    _store_halves(hw_ref, jnp.dot(h, W0_ref[...],
                                  preferred_element_type=jnp.float32) * dis)
    dis_ref[...] = dis


def _kmid_body(acc_ref, hw_ref, dis_ref, b_ref, g_ref, bb_ref, Wn_ref,
               out_ref):
    dis = dis_ref[...]
    t = _combine(acc_ref[...], hw_ref[...]) * dis + b_ref[...]
    h = _bn_relu(t, g_ref[...], bb_ref[...])
    _store_halves(out_ref, jnp.dot(h, Wn_ref[...],
                                   preferred_element_type=jnp.float32) * dis)


def _khead_body(acc_ref, hw_ref, dis_ref, b_ref, g_ref, bb_ref, batch_ref,
                fcW_ref, fcb_ref, fg_ref, fbb_ref, oW_ref, ob_ref, out_ref):
    dis = dis_ref[...]
    t = _combine(acc_ref[...], hw_ref[...]) * dis + b_ref[...]
    h = _bn_relu(t, g_ref[...], bb_ref[...])
    gid = lax.broadcasted_iota(jnp.int32, (NG, N), 0)
    oh = (gid == batch_ref[...]).astype(jnp.float32)
    g = jnp.dot(oh, h, preferred_element_type=jnp.float32)
    for i in range(3):
        g = jnp.dot(g, fcW_ref[i], preferred_element_type=jnp.float32) \
            + fcb_ref[i]
        g = _bn_relu(g, fg_ref[i], fbb_ref[i])
    logits = jnp.dot(g, oW_ref[...],
                     preferred_element_type=jnp.float32) + ob_ref[...]
    mx = jnp.max(logits, axis=1, keepdims=True)
    lse = jnp.log(jnp.sum(jnp.exp(logits - mx), axis=1, keepdims=True)) + mx
    out_ref[...] = logits - lse


@functools.cache
def _build():
    mesh = plsc.VectorSubcoreMesh(core_axis_name="c", subcore_axis_name="s")
    sc_deg = pl.kernel(
        _deg_body,
        out_type=jax.ShapeDtypeStruct((NC, NP, DW), jnp.float32),
        mesh=mesh,
        scratch_types=[
            pltpu.VMEM((NCHD, CH), jnp.int32),
            pltpu.VMEM((CH, DW), jnp.float32),
            pltpu.VMEM((RPT, DW), jnp.float32),
            pltpu.VMEM_SHARED((NP, DW), jnp.float32),
        ],
        name="sc_deg",
        compiler_params=pltpu.CompilerParams(use_tc_tiling_on_sc=False),
    )
    sc_msg = pl.kernel(
        _msg_body,
        out_type=jax.ShapeDtypeStruct((NC, NP, HH), jnp.float32),
        mesh=mesh,
        scratch_types=[
            pltpu.VMEM((NCH, CH), jnp.int32),
            pltpu.VMEM((NCH, CH), jnp.int32),
            pltpu.VMEM((CH, HH), jnp.float32),
            pltpu.VMEM((ZR, HH), jnp.float32),
            pltpu.VMEM_SHARED((NP, HH), jnp.float32),
            pltpu.SemaphoreType.DMA,
        ],
        name="sc_msg",
        compiler_params=pltpu.CompilerParams(use_tc_tiling_on_sc=False),
    )
    k0 = pl.pallas_call(
        _k0_body,
        out_shape=(jax.ShapeDtypeStruct((NC, N, HH), jnp.float32),
                   jax.ShapeDtypeStruct((N, 1), jnp.float32)),
    )
    kmid = pl.pallas_call(
        _kmid_body,
        out_shape=jax.ShapeDtypeStruct((NC, N, HH), jnp.float32),
    )
    khead = pl.pallas_call(
        _khead_body,
        out_shape=jax.ShapeDtypeStruct((NG, NOUT), jnp.float32),
    )
    return sc_deg, sc_msg, k0, kmid, khead


def kernel(x, edge_index, batch, atom_W, atom_b, conv_W, conv_b, norm_g,
           norm_b, fc_W, fc_b, fcn_g, fcn_b, out_W, out_b):
    sc_deg, sc_msg, k0, kmid, khead = _build()
    src_d = edge_index[0].reshape(NW, NCHD, CH)
    dst_d = edge_index[1].reshape(NW, NCHD, CH)
    src_m = edge_index[0].reshape(NS, NCH, CH)
    dst_m = edge_index[1].reshape(NS, NCH, CH)
    degp = sc_deg(dst_d)
    hw, dis = k0(x, atom_W, atom_b.reshape(1, H), conv_W[0], degp)
    out = None
    for i in range(3):
        acc = sc_msg(hw, src_m, dst_m)
        if i < 2:
            hw = kmid(acc, hw, dis, conv_b[i].reshape(1, H),
                      norm_g[i].reshape(1, H), norm_b[i].reshape(1, H),
                      conv_W[i + 1])
        else:
            out = khead(acc, hw, dis, conv_b[i].reshape(1, H),
                        norm_g[i].reshape(1, H), norm_b[i].reshape(1, H),
                        batch.reshape(1, N), fc_W, fc_b.reshape(3, 1, H),
                        fcn_g.reshape(3, 1, H), fcn_b.reshape(3, 1, H),
                        out_W, out_b.reshape(1, NOUT))
    return out


# speedup vs baseline: 22.6864x; 1.8626x over previous
import functools

import jax
import jax.numpy as jnp
from jax import lax
from jax.experimental import pallas as pl
from jax.experimental.pallas import tpu as pltpu
from jax.experimental.pallas import tpu_sc as plsc

N = 10000
E = 320000
H = 128
NG = 128
NOUT = 2
EPS = 1e-5

NC = 2
NS = 16
NW = NC * NS
HH = H // NC
EPS_T = E // NS
CH = 125
NCH = EPS_T // CH
EPW = E // NW
NCHD = EPW // CH
NP = 10240
RPT = NP // NS
ZR = 128
L = 16
DW = 16


def _deg_body(dst_hbm, degp_hbm, dst_v, ones_v, zb_v, degacc_sh):
    c = lax.axis_index("c")
    s = lax.axis_index("s")
    w = c * NS + s
    pltpu.sync_copy(dst_hbm.at[w], dst_v)

    ones = jnp.ones((L,), jnp.float32)

    def fill_ones(i, carry):
        ones_v[i, pl.ds(0, L)] = ones
        return carry

    lax.fori_loop(0, CH, fill_ones, 0)

    def fill_zero(i, carry):
        zb_v[i, pl.ds(0, L)] = jnp.zeros((L,), jnp.float32)
        return carry

    lax.fori_loop(0, RPT, fill_zero, 0)
    pltpu.sync_copy(zb_v, degacc_sh.at[pl.ds(s * RPT, RPT)])
    plsc.subcore_barrier()

    def chunk(ci, carry):
        pltpu.sync_copy(ones_v, degacc_sh.at[dst_v.at[ci]], add=True)
        return carry

    lax.fori_loop(0, NCHD, chunk, 0)
    plsc.subcore_barrier()
    pltpu.sync_copy(degacc_sh.at[pl.ds(s * RPT, RPT)],
                    degp_hbm.at[c, pl.ds(s * RPT, RPT)])


def _msg_body(hw_hbm, src_hbm, dst_hbm, out_hbm, src_v, dst_v, rows_a, rows_b,
              zb_v, acc_sh, sem_a, sem_b):
    c = lax.axis_index("c")
    s = lax.axis_index("s")

    def zero_row(i, carry):
        for j in range(HH // L):
            zb_v[i, pl.ds(j * L, L)] = jnp.zeros((L,), jnp.float32)
        return carry

    lax.fori_loop(0, ZR, zero_row, 0)
    for j in range(RPT // ZR):
        pltpu.sync_copy(zb_v, acc_sh.at[pl.ds(s * RPT + j * ZR, ZR)])

    pltpu.sync_copy(src_hbm.at[s], src_v)
    pltpu.sync_copy(dst_hbm.at[s], dst_v)
    plsc.subcore_barrier()

    hw_c = hw_hbm.at[c]
    pltpu.async_copy(hw_c.at[src_v.at[0]], rows_a, sem_a)

    def pair(k, carry):
        ca = 2 * k
        pltpu.async_copy(hw_c.at[src_v.at[ca + 1]], rows_b, sem_b)
        pltpu.make_async_copy(hw_c.at[src_v.at[ca]], rows_a, sem_a).wait()
        pltpu.sync_copy(rows_a, acc_sh.at[dst_v.at[ca]], add=True)
        nxt = jnp.minimum(ca + 2, NCH - 1)
        pltpu.async_copy(hw_c.at[src_v.at[nxt]], rows_a, sem_a)
        pltpu.make_async_copy(hw_c.at[src_v.at[ca + 1]], rows_b, sem_b).wait()
        pltpu.sync_copy(rows_b, acc_sh.at[dst_v.at[ca + 1]], add=True)
        return carry

    lax.fori_loop(0, NCH // 2, pair, 0)
    pltpu.make_async_copy(hw_c.at[src_v.at[NCH - 1]], rows_a, sem_a).wait()
    plsc.subcore_barrier()

    pltpu.sync_copy(acc_sh.at[pl.ds(s * RPT, RPT)],
                    out_hbm.at[c, pl.ds(s * RPT, RPT)])


def _elu(v):
    return jnp.where(v > 0, v, jnp.exp(v) - 1.0)


def _bn_relu(t, g, b):
    m = jnp.mean(t, axis=0, keepdims=True)
    v = jnp.mean((t - m) ** 2, axis=0, keepdims=True)
    return jnp.maximum((t - m) * lax.rsqrt(v + EPS) * g + b, 0.0)


def _store_halves(hw_ref, hww):
    hw_ref[0] = hww[:, :HH]
    hw_ref[1] = hww[:, HH:]


def _combine(a, hwp):
    return jnp.concatenate([a[0, :N] + hwp[0], a[1, :N] + hwp[1]], axis=1)


def _k0_body(x_ref, aW_ref, ab_ref, W0_ref, degp_ref, hw_ref, dis_ref):
    dp = degp_ref[...]
    deg = dp[0, :N, 0:1] + dp[1, :N, 0:1] + 1.0
    dis = lax.rsqrt(deg)
    h = _elu(jnp.dot(x_ref[...], aW_ref[...],
                     preferred_element_type=jnp.float32) + ab_ref[...])
    _store_halves(hw_ref, jnp.dot(h, W0_ref[...],
                                  preferred_element_type=jnp.float32) * dis)
    dis_ref[...] = dis


def _kmid_body(acc_ref, hw_ref, dis_ref, b_ref, g_ref, bb_ref, Wn_ref,
               out_ref):
    dis = dis_ref[...]
    t = _combine(acc_ref[...], hw_ref[...]) * dis + b_ref[...]
    h = _bn_relu(t, g_ref[...], bb_ref[...])
    _store_halves(out_ref, jnp.dot(h, Wn_ref[...],
                                   preferred_element_type=jnp.float32) * dis)


def _khead_body(acc_ref, hw_ref, dis_ref, b_ref, g_ref, bb_ref, batch_ref,
                fcW_ref, fcb_ref, fg_ref, fbb_ref, oW_ref, ob_ref, out_ref):
    dis = dis_ref[...]
    t = _combine(acc_ref[...], hw_ref[...]) * dis + b_ref[...]
    h = _bn_relu(t, g_ref[...], bb_ref[...])
    gid = lax.broadcasted_iota(jnp.int32, (NG, N), 0)
    oh = (gid == batch_ref[...]).astype(jnp.float32)
    g = jnp.dot(oh, h, preferred_element_type=jnp.float32)
    for i in range(3):
        g = jnp.dot(g, fcW_ref[i], preferred_element_type=jnp.float32) \
            + fcb_ref[i]
        g = _bn_relu(g, fg_ref[i], fbb_ref[i])
    logits = jnp.dot(g, oW_ref[...],
                     preferred_element_type=jnp.float32) + ob_ref[...]
    mx = jnp.max(logits, axis=1, keepdims=True)
    lse = jnp.log(jnp.sum(jnp.exp(logits - mx), axis=1, keepdims=True)) + mx
    out_ref[...] = logits - lse


@functools.cache
def _build():
    mesh = plsc.VectorSubcoreMesh(core_axis_name="c", subcore_axis_name="s")
    sc_deg = pl.kernel(
        _deg_body,
        out_type=jax.ShapeDtypeStruct((NC, NP, DW), jnp.float32),
        mesh=mesh,
        scratch_types=[
            pltpu.VMEM((NCHD, CH), jnp.int32),
            pltpu.VMEM((CH, DW), jnp.float32),
            pltpu.VMEM((RPT, DW), jnp.float32),
            pltpu.VMEM_SHARED((NP, DW), jnp.float32),
        ],
        name="sc_deg",
        compiler_params=pltpu.CompilerParams(use_tc_tiling_on_sc=False),
    )
    sc_msg = pl.kernel(
        _msg_body,
        out_type=jax.ShapeDtypeStruct((NC, NP, HH), jnp.float32),
        mesh=mesh,
        scratch_types=[
            pltpu.VMEM((NCH, CH), jnp.int32),
            pltpu.VMEM((NCH, CH), jnp.int32),
            pltpu.VMEM((CH, HH), jnp.float32),
            pltpu.VMEM((CH, HH), jnp.float32),
            pltpu.VMEM((ZR, HH), jnp.float32),
            pltpu.VMEM_SHARED((NP, HH), jnp.float32),
            pltpu.SemaphoreType.DMA,
            pltpu.SemaphoreType.DMA,
        ],
        name="sc_msg",
        compiler_params=pltpu.CompilerParams(use_tc_tiling_on_sc=False),
    )
    k0 = pl.pallas_call(
        _k0_body,
        out_shape=(jax.ShapeDtypeStruct((NC, N, HH), jnp.float32),
                   jax.ShapeDtypeStruct((N, 1), jnp.float32)),
    )
    kmid = pl.pallas_call(
        _kmid_body,
        out_shape=jax.ShapeDtypeStruct((NC, N, HH), jnp.float32),
    )
    khead = pl.pallas_call(
        _khead_body,
        out_shape=jax.ShapeDtypeStruct((NG, NOUT), jnp.float32),
    )
    return sc_deg, sc_msg, k0, kmid, khead


def kernel(x, edge_index, batch, atom_W, atom_b, conv_W, conv_b, norm_g,
           norm_b, fc_W, fc_b, fcn_g, fcn_b, out_W, out_b):
    sc_deg, sc_msg, k0, kmid, khead = _build()
    src_d = edge_index[0].reshape(NW, NCHD, CH)
    dst_d = edge_index[1].reshape(NW, NCHD, CH)
    src_m = edge_index[0].reshape(NS, NCH, CH)
    dst_m = edge_index[1].reshape(NS, NCH, CH)
    degp = sc_deg(dst_d)
    hw, dis = k0(x, atom_W, atom_b.reshape(1, H), conv_W[0], degp)
    out = None
    for i in range(3):
        acc = sc_msg(hw, src_m, dst_m)
        if i < 2:
            hw = kmid(acc, hw, dis, conv_b[i].reshape(1, H),
                      norm_g[i].reshape(1, H), norm_b[i].reshape(1, H),
                      conv_W[i + 1])
        else:
            out = khead(acc, hw, dis, conv_b[i].reshape(1, H),
                        norm_g[i].reshape(1, H), norm_b[i].reshape(1, H),
                        batch.reshape(1, N), fc_W, fc_b.reshape(3, 1, H),
                        fcn_g.reshape(3, 1, H), fcn_b.reshape(3, 1, H),
                        out_W, out_b.reshape(1, NOUT))
    return out


# speedup vs baseline: 27.2043x; 1.1991x over previous
import functools

import jax
import jax.numpy as jnp
from jax import lax
from jax.experimental import pallas as pl
from jax.experimental.pallas import tpu as pltpu
from jax.experimental.pallas import tpu_sc as plsc

N = 10000
E = 320000
H = 128
NG = 128
NOUT = 2
EPS = 1e-5

NC = 2
NS = 16
NW = NC * NS
HH = H // NC
EPS_T = E // NS
CH = 125
NCH = EPS_T // CH
EPW = E // NW
NCHD = EPW // CH
NP = 10240
RPT = NP // NS
ZR = 128
L = 16
NBUF = 5
DW = 16


def _deg_body(dst_hbm, degp_hbm, dst_v, ones_v, zb_v, degacc_sh, dsem):
    c = lax.axis_index("c")
    s = lax.axis_index("s")
    w = c * NS + s
    pltpu.sync_copy(dst_hbm.at[w], dst_v)

    ones = jnp.ones((L,), jnp.float32)

    def fill_ones(i, carry):
        ones_v[i, pl.ds(0, L)] = ones
        return carry

    lax.fori_loop(0, CH, fill_ones, 0)

    def fill_zero(i, carry):
        zb_v[i, pl.ds(0, L)] = jnp.zeros((L,), jnp.float32)
        return carry

    lax.fori_loop(0, RPT, fill_zero, 0)
    pltpu.sync_copy(zb_v, degacc_sh.at[pl.ds(s * RPT, RPT)])
    plsc.subcore_barrier()

    def chunk(ci, carry):
        pltpu.async_copy(ones_v, degacc_sh.at[dst_v.at[ci]], dsem, add=True)
        return carry

    lax.fori_loop(0, NCHD, chunk, 0)

    def drain(ci, carry):
        pltpu.make_async_copy(ones_v, degacc_sh.at[dst_v.at[ci]], dsem).wait()
        return carry

    lax.fori_loop(0, NCHD, drain, 0)
    plsc.subcore_barrier()
    pltpu.sync_copy(degacc_sh.at[pl.ds(s * RPT, RPT)],
                    degp_hbm.at[c, pl.ds(s * RPT, RPT)])


def _msg_body(hw_hbm, src_hbm, dst_hbm, out_hbm, src_v, dst_v, rows_a, rows_b,
              rows_c, rows_d, rows_e, zb_v, acc_sh, gsem_a, gsem_b,
              gsem_c, gsem_d, gsem_e, ssem_a, ssem_b, ssem_c, ssem_d,
              ssem_e, psem_a, psem_b):
    c = lax.axis_index("c")
    s = lax.axis_index("s")

    pltpu.async_copy(src_hbm.at[s], src_v, psem_a)
    pltpu.async_copy(dst_hbm.at[s], dst_v, psem_b)

    def zero_row(i, carry):
        for j in range(HH // L):
            zb_v[i, pl.ds(j * L, L)] = jnp.zeros((L,), jnp.float32)
        return carry

    lax.fori_loop(0, ZR, zero_row, 0)
    for j in range(RPT // ZR):
        pltpu.sync_copy(zb_v, acc_sh.at[pl.ds(s * RPT + j * ZR, ZR)])

    hw_c = hw_hbm.at[c]
    bufs = (rows_a, rows_b, rows_c, rows_d, rows_e)
    gsems = (gsem_a, gsem_b, gsem_c, gsem_d, gsem_e)
    ssems = (ssem_a, ssem_b, ssem_c, ssem_d, ssem_e)
    pltpu.make_async_copy(src_hbm.at[s], src_v, psem_a).wait()
    pltpu.make_async_copy(dst_hbm.at[s], dst_v, psem_b).wait()
    for j in range(NBUF):
        pltpu.async_copy(hw_c.at[src_v.at[j]], bufs[j], gsems[j])
    plsc.subcore_barrier()

    def quad(k, carry):
        base = NBUF * k
        for j in range(NBUF):
            ci = base + j
            pltpu.make_async_copy(hw_c.at[src_v.at[ci]], bufs[j],
                                  gsems[j]).wait()
            pltpu.async_copy(bufs[j], acc_sh.at[dst_v.at[ci]], ssems[j],
                             add=True)
        for j in range(NBUF):
            ci = base + j
            pltpu.make_async_copy(bufs[j], acc_sh.at[dst_v.at[ci]],
                                  ssems[j]).wait()
            nxt = jnp.minimum(ci + NBUF, NCH - 1)
            pltpu.async_copy(hw_c.at[src_v.at[nxt]], bufs[j], gsems[j])
        return carry

    lax.fori_loop(0, NCH // NBUF, quad, 0)
    for j in range(NBUF):
        pltpu.make_async_copy(hw_c.at[src_v.at[NCH - 1]], bufs[j],
                              gsems[j]).wait()
    plsc.subcore_barrier()

    pltpu.sync_copy(acc_sh.at[pl.ds(s * RPT, RPT)],
                    out_hbm.at[pl.ds(s * RPT, RPT), pl.ds(c * HH, HH)])


def _elu(v):
    return jnp.where(v > 0, v, jnp.exp(v) - 1.0)


def _bn_relu(t, g, b):
    m = jnp.mean(t, axis=0, keepdims=True)
    v = jnp.mean((t - m) ** 2, axis=0, keepdims=True)
    return jnp.maximum((t - m) * lax.rsqrt(v + EPS) * g + b, 0.0)


def _store_halves(hw_ref, hww):
    hw_ref[0] = hww[:, :HH]
    hw_ref[1] = hww[:, HH:]


def _k0_body(x_ref, aW_ref, ab_ref, W0_ref, degp_ref, hw_ref, dis_ref):
    dp = degp_ref[...]
    deg = dp[0, :N, 0:1] + dp[1, :N, 0:1] + 1.0
    dis = lax.rsqrt(deg)
    h = _elu(jnp.dot(x_ref[...], aW_ref[...],
                     preferred_element_type=jnp.float32) + ab_ref[...])
    _store_halves(hw_ref, jnp.dot(h, W0_ref[...],
                                  preferred_element_type=jnp.float32) * dis)
    dis_ref[...] = dis


def _kmid_body(acc_ref, hw_ref, dis_ref, b_ref, g_ref, bb_ref, Wn_ref,
               out_ref):
    dis = dis_ref[...]
    hwp = hw_ref[...]
    t = (acc_ref[...][:N] + jnp.concatenate([hwp[0], hwp[1]], axis=1)) \
        * dis + b_ref[...]
    h = _bn_relu(t, g_ref[...], bb_ref[...])
    _store_halves(out_ref, jnp.dot(h, Wn_ref[...],
                                   preferred_element_type=jnp.float32) * dis)


def _khead_body(acc_ref, hw_ref, dis_ref, b_ref, g_ref, bb_ref, batch_ref,
                fcW_ref, fcb_ref, fg_ref, fbb_ref, oW_ref, ob_ref, out_ref):
    dis = dis_ref[...]
    hwp = hw_ref[...]
    t = (acc_ref[...][:N] + jnp.concatenate([hwp[0], hwp[1]], axis=1)) \
        * dis + b_ref[...]
    h = _bn_relu(t, g_ref[...], bb_ref[...])
    gid = lax.broadcasted_iota(jnp.int32, (NG, N), 0)
    oh = (gid == batch_ref[...]).astype(jnp.float32)
    g = jnp.dot(oh, h, preferred_element_type=jnp.float32)
    for i in range(3):
        g = jnp.dot(g, fcW_ref[i], preferred_element_type=jnp.float32) \
            + fcb_ref[i]
        g = _bn_relu(g, fg_ref[i], fbb_ref[i])
    logits = jnp.dot(g, oW_ref[...],
                     preferred_element_type=jnp.float32) + ob_ref[...]
    mx = jnp.max(logits, axis=1, keepdims=True)
    lse = jnp.log(jnp.sum(jnp.exp(logits - mx), axis=1, keepdims=True)) + mx
    out_ref[...] = logits - lse


@functools.cache
def _build():
    mesh = plsc.VectorSubcoreMesh(core_axis_name="c", subcore_axis_name="s")
    sc_deg = pl.kernel(
        _deg_body,
        out_type=jax.ShapeDtypeStruct((NC, NP, DW), jnp.float32),
        mesh=mesh,
        scratch_types=[
            pltpu.VMEM((NCHD, CH), jnp.int32),
            pltpu.VMEM((CH, DW), jnp.float32),
            pltpu.VMEM((RPT, DW), jnp.float32),
            pltpu.VMEM_SHARED((NP, DW), jnp.float32),
            pltpu.SemaphoreType.DMA,
        ],
        name="sc_deg",
        compiler_params=pltpu.CompilerParams(use_tc_tiling_on_sc=False),
    )
    sc_msg = pl.kernel(
        _msg_body,
        out_type=jax.ShapeDtypeStruct((NP, H), jnp.float32),
        mesh=mesh,
        scratch_types=[
            pltpu.VMEM((NCH, CH), jnp.int32),
            pltpu.VMEM((NCH, CH), jnp.int32),
            pltpu.VMEM((CH, HH), jnp.float32),
            pltpu.VMEM((CH, HH), jnp.float32),
            pltpu.VMEM((CH, HH), jnp.float32),
            pltpu.VMEM((CH, HH), jnp.float32),
            pltpu.VMEM((CH, HH), jnp.float32),
            pltpu.VMEM((ZR, HH), jnp.float32),
            pltpu.VMEM_SHARED((NP, HH), jnp.float32),
        ] + [pltpu.SemaphoreType.DMA] * 12,
        name="sc_msg",
        compiler_params=pltpu.CompilerParams(use_tc_tiling_on_sc=False),
    )
    k0 = pl.pallas_call(
        _k0_body,
        out_shape=(jax.ShapeDtypeStruct((NC, N, HH), jnp.float32),
                   jax.ShapeDtypeStruct((N, 1), jnp.float32)),
    )
    kmid = pl.pallas_call(
        _kmid_body,
        out_shape=jax.ShapeDtypeStruct((NC, N, HH), jnp.float32),
    )
    khead = pl.pallas_call(
        _khead_body,
        out_shape=jax.ShapeDtypeStruct((NG, NOUT), jnp.float32),
    )
    return sc_deg, sc_msg, k0, kmid, khead


def kernel(x, edge_index, batch, atom_W, atom_b, conv_W, conv_b, norm_g,
           norm_b, fc_W, fc_b, fcn_g, fcn_b, out_W, out_b):
    sc_deg, sc_msg, k0, kmid, khead = _build()
    src_d = edge_index[0].reshape(NW, NCHD, CH)
    dst_d = edge_index[1].reshape(NW, NCHD, CH)
    src_m = edge_index[0].reshape(NS, NCH, CH)
    dst_m = edge_index[1].reshape(NS, NCH, CH)
    degp = sc_deg(dst_d)
    hw, dis = k0(x, atom_W, atom_b.reshape(1, H), conv_W[0], degp)
    out = None
    for i in range(3):
        acc = sc_msg(hw, src_m, dst_m)
        if i < 2:
            hw = kmid(acc, hw, dis, conv_b[i].reshape(1, H),
                      norm_g[i].reshape(1, H), norm_b[i].reshape(1, H),
                      conv_W[i + 1])
        else:
            out = khead(acc, hw, dis, conv_b[i].reshape(1, H),
                        norm_g[i].reshape(1, H), norm_b[i].reshape(1, H),
                        batch.reshape(1, N), fc_W, fc_b.reshape(3, 1, H),
                        fcn_g.reshape(3, 1, H), fcn_b.reshape(3, 1, H),
                        out_W, out_b.reshape(1, NOUT))
    return out
